# Initial kernel scaffold; baseline (speedup 1.0000x reference)
import jax
import jax.numpy as jnp
from jax.experimental import pallas as pl


def kernel(states_action, states_graph_ids, states_first, states_second, ordered_edges, W0, b0, att_kernels, att_attn, Wr1, br1, Wr2, br2, Wr3, br3):
    raise NotImplementedError("write your pallas kernel here")


# speedup vs baseline: 51.1540x; 51.1540x over previous
import functools

import jax
import jax.numpy as jnp
from jax import lax
from jax.experimental import pallas as pl
from jax.experimental.pallas import tpu as pltpu
from jax.experimental.pallas import tpu_sc as plsc

N = 10000
E = 320000
NC = 2
NS = 16
NW = NC * NS
EPW = E // NW
CH = 80
NCHA = EPW // CH
EPT = E // NS
NCHC = EPT // CH
NP = 10112
RPT = NP // NS

_mesh = plsc.VectorSubcoreMesh(core_axis_name="c", subcore_axis_name="s")


def _pre_body(s_ref, w0_ref, b0_ref, kcat_ref, kas_ref, kad_ref,
              x0_ref, nst_ref, apad_ref):
    s = s_ref[...]
    nst = jnp.dot(s, kcat_ref[...], preferred_element_type=jnp.float32)
    nst_ref[...] = nst
    x0_ref[...] = jnp.maximum(
        jnp.dot(s, w0_ref[...], preferred_element_type=jnp.float32) + b0_ref[...], 0.0)
    blk = (lax.broadcasted_iota(jnp.int32, (256, 16), 0) // 16
           == lax.broadcasted_iota(jnp.int32, (256, 16), 1)).astype(jnp.float32)
    a_src = jnp.dot(nst * kas_ref[...], blk, preferred_element_type=jnp.float32)
    a_dst = jnp.dot(nst * kad_ref[...], blk, preferred_element_type=jnp.float32)
    apad_ref[...] = jnp.zeros((NP, 128), jnp.float32)
    apad_ref[0:N, 0:16] = a_src
    apad_ref[0:N, 16:32] = a_dst


def _pre(s, w0, b0, kcat, kas, kad):
    return pl.pallas_call(
        _pre_body,
        out_shape=[
            jax.ShapeDtypeStruct((N, 128), jnp.float32),
            jax.ShapeDtypeStruct((N, 256), jnp.float32),
            jax.ShapeDtypeStruct((NP, 128), jnp.float32),
        ],
    )(s, w0, b0, kcat, kas, kad)


def _selu(t):
    return 1.0507009873554805 * jnp.where(t > 0, t, 1.6732632423543772 * (jnp.exp(t) - 1.0))


def _fin_body(x0_ref, a0_ref, a1_ref, d0_ref, d1_ref, gid_ref, wr1_ref,
              br1_ref, wr2_ref, br2_ref, wr3_ref, br3_ref, out_ref):
    den = d0_ref[...] + d1_ref[...]
    hh = lax.broadcasted_iota(jnp.int32, (16, 128), 0)
    uu = lax.broadcasted_iota(jnp.int32, (16, 128), 1) // 16
    e0 = (hh == uu).astype(jnp.float32)
    e1 = (hh == uu + 8).astype(jnp.float32)
    dl0 = jnp.dot(den, e0, preferred_element_type=jnp.float32)
    dl1 = jnp.dot(den, e1, preferred_element_type=jnp.float32)
    dl0 = jnp.maximum(dl0, 1e-20)
    dl1 = jnp.maximum(dl1, 1e-20)
    x = (x0_ref[...] + jnp.maximum(a0_ref[...] / dl0, 0.0)
         + jnp.maximum(a1_ref[...] / dl1, 0.0))
    oh = (lax.broadcasted_iota(jnp.int32, (64, N), 0) == gid_ref[...]).astype(jnp.float32)
    pooled = jnp.dot(oh, x, preferred_element_type=jnp.float32)
    h = _selu(jnp.dot(pooled, wr1_ref[...], preferred_element_type=jnp.float32) + br1_ref[...])
    h = _selu(jnp.dot(h, wr2_ref[...], preferred_element_type=jnp.float32) + br2_ref[...])
    out_ref[...] = jnp.dot(h, wr3_ref[...], preferred_element_type=jnp.float32) + br3_ref[...]


def _fin(x0, a0, a1, d0, d1, gid, wr1, br1, wr2, br2, wr3, br3):
    return pl.pallas_call(
        _fin_body,
        out_shape=jax.ShapeDtypeStruct((64, 1), jnp.float32),
    )(x0, a0, a1, d0, d1, gid, wr1, br1, wr2, br2, wr3, br3)


@functools.partial(
    pl.kernel, mesh=_mesh,
    out_type=[
        jax.ShapeDtypeStruct((E, 16), jnp.float32),
        jax.ShapeDtypeStruct((2 * NP, 128), jnp.float32),
    ],
    scratch_types=[
        pltpu.VMEM((CH,), jnp.int32),
        pltpu.VMEM((CH,), jnp.int32),
        pltpu.VMEM((CH, 128), jnp.float32),
        pltpu.VMEM((CH, 128), jnp.float32),
        pltpu.VMEM((CH, 128), jnp.float32),
        pltpu.VMEM((CH, 16), jnp.float32),
        pltpu.VMEM_SHARED((NP, 128), jnp.float32),
        pltpu.SemaphoreType.DMA,
        pltpu.SemaphoreType.DMA,
    ],
)
def _pass_a(src_hbm, dst_hbm, apad_hbm, slin_hbm, dden_hbm,
            sidx, didx, asr, adr, spad, sv, dsh, sem1, sem2):
    cid = lax.axis_index("c")
    sid = lax.axis_index("s")
    wid = sid * NC + cid

    def zloop(i, _):
        for k in range(8):
            spad[i, pl.ds(k * 16, 16)] = jnp.zeros((16,), jnp.float32)
        return 0
    lax.fori_loop(0, CH, zloop, 0)
    for j in range(7):
        pltpu.sync_copy(spad, dsh.at[pl.ds(sid * RPT + j * 80, 80)])
    pltpu.sync_copy(spad.at[pl.ds(0, 72)], dsh.at[pl.ds(sid * RPT + 560, 72)])
    plsc.subcore_barrier()

    def chunk(c, _):
        base = wid * EPW + c * CH
        pltpu.sync_copy(src_hbm.at[pl.ds(base, CH)], sidx)
        pltpu.sync_copy(dst_hbm.at[pl.ds(base, CH)], didx)
        cp1 = pltpu.async_copy(apad_hbm.at[sidx], asr, sem1)
        cp2 = pltpu.async_copy(apad_hbm.at[didx], adr, sem2)
        cp1.wait()
        cp2.wait()

        def body(i, _):
            v = asr[i, pl.ds(0, 16)] + adr[i, pl.ds(16, 16)]
            v = jnp.where(v >= 0.0, v, 0.2 * v)
            v = jnp.clip(v, -2.0, 2.0)
            s = jnp.exp(v)
            sv[i, :] = s
            spad[i, pl.ds(0, 16)] = s
            return 0
        lax.fori_loop(0, CH, body, 0)
        pltpu.sync_copy(sv, slin_hbm.at[pl.ds(base, CH)])
        pltpu.sync_copy(spad, dsh.at[sidx], add=True)
        return 0
    lax.fori_loop(0, NCHA, chunk, 0)

    plsc.subcore_barrier()
    pltpu.sync_copy(dsh.at[pl.ds(sid * RPT, RPT)],
                    dden_hbm.at[pl.ds(cid * NP + sid * RPT, RPT)])


@functools.partial(
    pl.kernel, mesh=_mesh,
    out_type=jax.ShapeDtypeStruct((2 * NP, 128), jnp.float32),
    scratch_types=[
        pltpu.VMEM((CH,), jnp.int32),
        pltpu.VMEM((CH,), jnp.int32),
        pltpu.VMEM((CH,), jnp.int32),
        pltpu.VMEM((CH, 16), jnp.float32),
        pltpu.VMEM((CH, 128), jnp.float32),
        pltpu.VMEM_SHARED((NP, 128), jnp.float32),
        pltpu.SemaphoreType.DMA,
        pltpu.SemaphoreType.DMA,
    ],
)
def _pass_c(src_hbm, dst_hbm, slin_hbm, nst2_hbm, out_hbm,
            sidx, didx, idxc, sv, rows, accsh, sem1, sem2):
    cid = lax.axis_index("c")
    sid = lax.axis_index("s")

    def zloop(i, _):
        for k in range(8):
            rows[i, pl.ds(k * 16, 16)] = jnp.zeros((16,), jnp.float32)
        return 0
    lax.fori_loop(0, CH, zloop, 0)
    for j in range(7):
        pltpu.sync_copy(rows, accsh.at[pl.ds(sid * RPT + j * 80, 80)])
    pltpu.sync_copy(rows.at[pl.ds(0, 72)], accsh.at[pl.ds(sid * RPT + 560, 72)])
    plsc.subcore_barrier()

    is_c0 = cid == 0
    off = cid * NP

    def chunk(c, _):
        base = sid * EPT + c * CH
        pltpu.sync_copy(src_hbm.at[pl.ds(base, CH)], sidx)
        pltpu.sync_copy(dst_hbm.at[pl.ds(base, CH)], didx)
        for k in range(CH // 16):
            idxc[pl.ds(k * 16, 16)] = didx[pl.ds(k * 16, 16)] + off
        cp1 = pltpu.async_copy(slin_hbm.at[pl.ds(base, CH)], sv, sem1)
        cp2 = pltpu.async_copy(nst2_hbm.at[idxc], rows, sem2)
        cp1.wait()
        cp2.wait()

        def body(i, _):
            srow = sv[i, :]
            for h in range(8):
                wsc = jnp.where(is_c0, srow[h], srow[8 + h])
                rows[i, pl.ds(h * 16, 16)] = rows[i, pl.ds(h * 16, 16)] * wsc
            return 0
        lax.fori_loop(0, CH, body, 0)
        pltpu.sync_copy(rows, accsh.at[sidx], add=True)
        return 0
    lax.fori_loop(0, NCHC, chunk, 0)

    plsc.subcore_barrier()
    pltpu.sync_copy(accsh.at[pl.ds(sid * RPT, RPT)],
                    out_hbm.at[pl.ds(cid * NP + sid * RPT, RPT)])


def kernel(states_action, states_graph_ids, states_first, states_second,
           ordered_edges, W0, b0, att_kernels, att_attn,
           Wr1, br1, Wr2, br2, Wr3, br3):
    f32 = jnp.float32
    kcat = jnp.transpose(att_kernels, (2, 0, 1, 3)).reshape(128, 256)
    kas = att_attn[:, :, :16, 0].reshape(1, 256).astype(f32)
    kad = att_attn[:, :, 16:, 0].reshape(1, 256).astype(f32)

    x0, nst_all, apad = _pre(states_action, W0, b0.reshape(1, 128), kcat, kas, kad)

    src = states_first.astype(jnp.int32)
    dst = states_second.astype(jnp.int32)

    slin, dden = _pass_a(src, dst, apad)
    pad = jnp.zeros((NP - N, 128), f32)
    nst2 = jnp.concatenate([nst_all[:, :128], pad, nst_all[:, 128:], pad], axis=0)
    acc = _pass_c(src, dst, slin, nst2)

    gid = states_graph_ids.astype(jnp.int32).reshape(1, N)
    return _fin(x0, acc[:N], acc[NP:NP + N],
                dden[:N, :16], dden[NP:NP + N, :16], gid,
                Wr1, br1.reshape(1, 35), Wr2, br2.reshape(1, 35),
                Wr3, br3.reshape(1, 1))


# speedup vs baseline: 87.5310x; 1.7111x over previous
import functools

import jax
import jax.numpy as jnp
from jax import lax
from jax.experimental import pallas as pl
from jax.experimental.pallas import tpu as pltpu
from jax.experimental.pallas import tpu_sc as plsc

N = 10000
E = 320000
NC = 2
NS = 16
NW = NC * NS
EPW = E // NW
CH = 80
NCHA = EPW // CH
EPT = E // NS
NCHC = EPT // CH
NP = 10112
RPT = NP // NS
NP8 = 1280
RP8 = NP8 // NS

_mesh = plsc.VectorSubcoreMesh(core_axis_name="c", subcore_axis_name="s")


def _pre_body(s_ref, w0_ref, b0_ref, kcat_ref, kas_ref, kad_ref,
              x0_ref, nst_ref, apad_ref):
    s = s_ref[...]
    nst = jnp.dot(s, kcat_ref[...], preferred_element_type=jnp.float32)
    nst_ref[...] = nst
    x0_ref[...] = jnp.maximum(
        jnp.dot(s, w0_ref[...], preferred_element_type=jnp.float32) + b0_ref[...], 0.0)
    blk = (lax.broadcasted_iota(jnp.int32, (256, 16), 0) // 16
           == lax.broadcasted_iota(jnp.int32, (256, 16), 1)).astype(jnp.float32)
    a_src = jnp.dot(nst * kas_ref[...], blk, preferred_element_type=jnp.float32)
    a_dst = jnp.dot(nst * kad_ref[...], blk, preferred_element_type=jnp.float32)
    apad_ref[...] = jnp.zeros((NP, 128), jnp.float32)
    apad_ref[0:N, 0:16] = a_src
    apad_ref[0:N, 16:32] = a_dst


def _pre(s, w0, b0, kcat, kas, kad):
    return pl.pallas_call(
        _pre_body,
        out_shape=[
            jax.ShapeDtypeStruct((N, 128), jnp.float32),
            jax.ShapeDtypeStruct((N, 256), jnp.float32),
            jax.ShapeDtypeStruct((NP, 128), jnp.float32),
        ],
    )(s, w0, b0, kcat, kas, kad)


def _selu(t):
    return 1.0507009873554805 * jnp.where(t > 0, t, 1.6732632423543772 * (jnp.exp(t) - 1.0))


def _fin_body(x0_ref, a0_ref, a1_ref, d0_ref, d1_ref, gid_ref, wr1_ref,
              br1_ref, wr2_ref, br2_ref, wr3_ref, br3_ref, out_ref):
    den = d0_ref[...] + d1_ref[...]
    hh = lax.broadcasted_iota(jnp.int32, (16, 128), 0)
    uu = lax.broadcasted_iota(jnp.int32, (16, 128), 1) // 16
    e0 = (hh == uu).astype(jnp.float32)
    e1 = (hh == uu + 8).astype(jnp.float32)
    dl0 = jnp.maximum(jnp.dot(den, e0, preferred_element_type=jnp.float32), 1e-20)
    dl1 = jnp.maximum(jnp.dot(den, e1, preferred_element_type=jnp.float32), 1e-20)
    x = (x0_ref[...] + jnp.maximum(a0_ref[...] / dl0, 0.0)
         + jnp.maximum(a1_ref[...] / dl1, 0.0))
    oh = (lax.broadcasted_iota(jnp.int32, (64, N), 0) == gid_ref[...]).astype(jnp.float32)
    pooled = jnp.dot(oh, x, preferred_element_type=jnp.float32)
    h = _selu(jnp.dot(pooled, wr1_ref[...], preferred_element_type=jnp.float32) + br1_ref[...])
    h = _selu(jnp.dot(h, wr2_ref[...], preferred_element_type=jnp.float32) + br2_ref[...])
    out_ref[...] = jnp.dot(h, wr3_ref[...], preferred_element_type=jnp.float32) + br3_ref[...]


def _fin(x0, a0, a1, d0, d1, gid, wr1, br1, wr2, br2, wr3, br3):
    return pl.pallas_call(
        _fin_body,
        out_shape=jax.ShapeDtypeStruct((64, 1), jnp.float32),
    )(x0, a0, a1, d0, d1, gid, wr1, br1, wr2, br2, wr3, br3)


@functools.partial(
    pl.kernel, mesh=_mesh,
    out_type=[
        jax.ShapeDtypeStruct((E, 16), jnp.float32),
        jax.ShapeDtypeStruct((2 * NP8, 128), jnp.float32),
    ],
    scratch_types=[
        pltpu.VMEM((EPW,), jnp.int32),
        pltpu.VMEM((EPW,), jnp.int32),
        pltpu.VMEM((CH,), jnp.int32),
        pltpu.VMEM((CH,), jnp.int32),
        pltpu.VMEM((CH,), jnp.int32),
        pltpu.VMEM((CH,), jnp.int32),
        pltpu.VMEM((CH,), jnp.int32),
        pltpu.VMEM((CH,), jnp.int32),
        pltpu.VMEM((CH,), jnp.int32),
        pltpu.VMEM((CH,), jnp.int32),
        pltpu.VMEM((CH, 128), jnp.float32),
        pltpu.VMEM((CH, 128), jnp.float32),
        pltpu.VMEM((CH, 128), jnp.float32),
        pltpu.VMEM((CH, 128), jnp.float32),
        pltpu.VMEM((CH, 16), jnp.float32),
        pltpu.VMEM((CH, 128), jnp.float32),
        pltpu.VMEM_SHARED((NP8, 128), jnp.float32),
        pltpu.SemaphoreType.DMA,
        pltpu.SemaphoreType.DMA,
        pltpu.SemaphoreType.DMA,
        pltpu.SemaphoreType.DMA,
    ],
)
def _pass_a(src_hbm, dst_hbm, apad_hbm, slin_hbm, dden_hbm,
            srcall, dstall, si0, si1, di0, di1, p0, p1, m0, m1,
            as0, as1, ad0, ad1, sv, spad, dsh, smA0, smA1, smB0, smB1):
    cid = lax.axis_index("c")
    sid = lax.axis_index("s")
    wid = sid * NC + cid
    tbase = wid * EPW

    def zloop(i, _):
        for k in range(8):
            spad[i, pl.ds(k * 16, 16)] = jnp.zeros((16,), jnp.float32)
        return 0
    lax.fori_loop(0, CH, zloop, 0)
    pltpu.sync_copy(spad, dsh.at[pl.ds(sid * RP8, RP8)])
    pltpu.sync_copy(src_hbm.at[pl.ds(tbase, EPW)], srcall)
    pltpu.sync_copy(dst_hbm.at[pl.ds(tbase, EPW)], dstall)
    plsc.subcore_barrier()

    slots = ((si0, di0, p0, m0, as0, ad0, smA0, smB0),
             (si1, di1, p1, m1, as1, ad1, smA1, smB1))

    def issue(c, b):
        si, di, pp, mm, asv, adv, smA, smB = slots[b]
        s0 = c * CH
        for k in range(CH // 16):
            sl = pl.ds(k * 16, 16)
            s16 = srcall[pl.ds(s0 + k * 16, 16)]
            si[sl] = s16
            pp[sl] = lax.shift_right_logical(s16, 3)
            mm[sl] = lax.bitwise_and(s16, 7)
            di[sl] = dstall[pl.ds(s0 + k * 16, 16)]
        pltpu.async_copy(apad_hbm.at[si], asv, smA)
        pltpu.async_copy(apad_hbm.at[di], adv, smB)

    def finish(c, b):
        si, di, pp, mm, asv, adv, smA, smB = slots[b]
        pltpu.make_async_copy(apad_hbm.at[si], asv, smA).wait()
        pltpu.make_async_copy(apad_hbm.at[di], adv, smB).wait()
        z16 = jnp.zeros((16,), jnp.float32)

        def body(j, _):
            mrow = mm[pl.ds(j * 16, 16)]
            for l in range(16):
                i = j * 16 + l
                v = asv[i, pl.ds(0, 16)] + adv[i, pl.ds(16, 16)]
                v = jnp.where(v >= 0.0, v, 0.2 * v)
                v = jnp.clip(v, -2.0, 2.0)
                s = jnp.exp(v)
                sv[i, :] = s
                mk = mrow[l]
                for k in range(8):
                    spad[i, pl.ds(k * 16, 16)] = jnp.where(mk == k, s, z16)
            return 0
        lax.fori_loop(0, CH // 16, body, 0)
        pltpu.sync_copy(sv, slin_hbm.at[pl.ds(tbase + c * CH, CH)])
        pltpu.sync_copy(spad, dsh.at[pp], add=True)

    issue(0, 0)

    def outer(g, _):
        c0 = 2 * g

        @pl.when(c0 + 1 < NCHA)
        def _():
            issue(c0 + 1, 1)
        finish(c0, 0)

        @pl.when(c0 + 2 < NCHA)
        def _():
            issue(c0 + 2, 0)

        @pl.when(c0 + 1 < NCHA)
        def _():
            finish(c0 + 1, 1)
        return 0
    lax.fori_loop(0, (NCHA + 1) // 2, outer, 0)

    plsc.subcore_barrier()
    pltpu.sync_copy(dsh.at[pl.ds(sid * RP8, RP8)],
                    dden_hbm.at[pl.ds(cid * NP8 + sid * RP8, RP8)])


@functools.partial(
    pl.kernel, mesh=_mesh,
    out_type=jax.ShapeDtypeStruct((2 * NP, 128), jnp.float32),
    scratch_types=[
        pltpu.VMEM((CH,), jnp.int32),
        pltpu.VMEM((CH,), jnp.int32),
        pltpu.VMEM((CH,), jnp.int32),
        pltpu.VMEM((CH,), jnp.int32),
        pltpu.VMEM((CH, 16), jnp.float32),
        pltpu.VMEM((CH, 16), jnp.float32),
        pltpu.VMEM((CH, 128), jnp.float32),
        pltpu.VMEM((CH, 128), jnp.float32),
        pltpu.VMEM_SHARED((NP, 128), jnp.float32),
        pltpu.SemaphoreType.DMA,
        pltpu.SemaphoreType.DMA,
        pltpu.SemaphoreType.DMA,
        pltpu.SemaphoreType.DMA,
    ],
)
def _pass_c(src_hbm, dst_hbm, slin_hbm, nst2_hbm, out_hbm,
            si0, si1, gi0, gi1, sv0, sv1, rw0, rw1,
            accsh, smS0, smS1, smG0, smG1):
    cid = lax.axis_index("c")
    sid = lax.axis_index("s")
    off = cid * NP
    tbase = sid * EPT

    def zloop(i, _):
        for k in range(8):
            rw0[i, pl.ds(k * 16, 16)] = jnp.zeros((16,), jnp.float32)
        return 0
    lax.fori_loop(0, CH, zloop, 0)
    for j in range(7):
        pltpu.sync_copy(rw0, accsh.at[pl.ds(sid * RPT + j * 80, 80)])
    pltpu.sync_copy(rw0.at[pl.ds(0, 72)], accsh.at[pl.ds(sid * RPT + 560, 72)])
    plsc.subcore_barrier()

    is_c0 = cid == 0
    slots = ((si0, gi0, sv0, rw0, smS0, smG0), (si1, gi1, sv1, rw1, smS1, smG1))

    def issue(c, b):
        si, gi, sv, rows, smS, smG = slots[b]
        s0 = tbase + c * CH
        pltpu.sync_copy(src_hbm.at[pl.ds(s0, CH)], si)
        pltpu.sync_copy(dst_hbm.at[pl.ds(s0, CH)], gi)
        for k in range(CH // 16):
            sl = pl.ds(k * 16, 16)
            gi[sl] = gi[sl] + off
        pltpu.async_copy(slin_hbm.at[pl.ds(s0, CH)], sv, smS)
        pltpu.async_copy(nst2_hbm.at[gi], rows, smG)

    def finish(c, b):
        si, gi, sv, rows, smS, smG = slots[b]
        pltpu.make_async_copy(slin_hbm.at[pl.ds(tbase + c * CH, CH)], sv, smS).wait()
        pltpu.make_async_copy(nst2_hbm.at[gi], rows, smG).wait()

        def body(i, _):
            srow = sv[i, :]
            for h in range(8):
                wsc = jnp.where(is_c0, srow[h], srow[8 + h])
                rows[i, pl.ds(h * 16, 16)] = rows[i, pl.ds(h * 16, 16)] * wsc
            return 0
        lax.fori_loop(0, CH, body, 0)
        pltpu.sync_copy(rows, accsh.at[si], add=True)

    issue(0, 0)

    def outer(g, _):
        c0 = 2 * g

        @pl.when(c0 + 1 < NCHC)
        def _():
            issue(c0 + 1, 1)
        finish(c0, 0)

        @pl.when(c0 + 2 < NCHC)
        def _():
            issue(c0 + 2, 0)

        @pl.when(c0 + 1 < NCHC)
        def _():
            finish(c0 + 1, 1)
        return 0
    lax.fori_loop(0, NCHC // 2, outer, 0)

    plsc.subcore_barrier()
    pltpu.sync_copy(accsh.at[pl.ds(sid * RPT, RPT)],
                    out_hbm.at[pl.ds(cid * NP + sid * RPT, RPT)])


def kernel(states_action, states_graph_ids, states_first, states_second,
           ordered_edges, W0, b0, att_kernels, att_attn,
           Wr1, br1, Wr2, br2, Wr3, br3):
    f32 = jnp.float32
    kcat = jnp.transpose(att_kernels, (2, 0, 1, 3)).reshape(128, 256)
    kas = att_attn[:, :, :16, 0].reshape(1, 256).astype(f32)
    kad = att_attn[:, :, 16:, 0].reshape(1, 256).astype(f32)

    x0, nst_all, apad = _pre(states_action, W0, b0.reshape(1, 128), kcat, kas, kad)

    src = states_first.astype(jnp.int32)
    dst = states_second.astype(jnp.int32)

    slin, dden = _pass_a(src, dst, apad)
    pad = jnp.zeros((NP - N, 128), f32)
    nst2 = jnp.concatenate([nst_all[:, :128], pad, nst_all[:, 128:], pad], axis=0)
    acc = _pass_c(src, dst, slin, nst2)

    gid = states_graph_ids.astype(jnp.int32).reshape(1, N)
    d0 = dden[:NP8].reshape(NP8 * 8, 16)[:N]
    d1 = dden[NP8:].reshape(NP8 * 8, 16)[:N]
    return _fin(x0, acc[:N], acc[NP:NP + N], d0, d1, gid,
                Wr1, br1.reshape(1, 35), Wr2, br2.reshape(1, 35),
                Wr3, br3.reshape(1, 1))


# speedup vs baseline: 108.1046x; 1.2350x over previous
import functools

import jax
import jax.numpy as jnp
from jax import lax
from jax.experimental import pallas as pl
from jax.experimental.pallas import tpu as pltpu
from jax.experimental.pallas import tpu_sc as plsc

N = 10000
E = 320000
NC = 2
NS = 16
NW = NC * NS
EPW = E // NW
CH = 80
NCHA = EPW // CH
EPT = E // NS
SEG = 4000
NSEG = EPT // SEG
NCHS = SEG // CH
NP = 10112
RPT = NP // NS
NP8 = 1280
RP8 = NP8 // NS

_mesh = plsc.VectorSubcoreMesh(core_axis_name="c", subcore_axis_name="s")


def _pre_body(s_ref, w0_ref, b0_ref, kcat_ref, kas_ref, kad_ref,
              x0_ref, nst_ref, apad_ref):
    s = s_ref[...]
    nst = jnp.dot(s, kcat_ref[...], preferred_element_type=jnp.float32)
    nst_ref[...] = nst
    x0_ref[...] = jnp.maximum(
        jnp.dot(s, w0_ref[...], preferred_element_type=jnp.float32) + b0_ref[...], 0.0)
    blk = (lax.broadcasted_iota(jnp.int32, (256, 16), 0) // 16
           == lax.broadcasted_iota(jnp.int32, (256, 16), 1)).astype(jnp.float32)
    a_src = jnp.dot(nst * kas_ref[...], blk, preferred_element_type=jnp.float32)
    a_dst = jnp.dot(nst * kad_ref[...], blk, preferred_element_type=jnp.float32)
    apad_ref[...] = jnp.zeros((NP, 128), jnp.float32)
    apad_ref[0:N, 0:16] = a_src
    apad_ref[0:N, 16:32] = a_dst


def _pre(s, w0, b0, kcat, kas, kad):
    return pl.pallas_call(
        _pre_body,
        out_shape=[
            jax.ShapeDtypeStruct((N, 128), jnp.float32),
            jax.ShapeDtypeStruct((N, 256), jnp.float32),
            jax.ShapeDtypeStruct((NP, 128), jnp.float32),
        ],
    )(s, w0, b0, kcat, kas, kad)


def _selu(t):
    return 1.0507009873554805 * jnp.where(t > 0, t, 1.6732632423543772 * (jnp.exp(t) - 1.0))


def _fin_body(x0_ref, a0_ref, a1_ref, d0_ref, d1_ref, gid_ref, wr1_ref,
              br1_ref, wr2_ref, br2_ref, wr3_ref, br3_ref, out_ref):
    den = d0_ref[...] + d1_ref[...]
    hh = lax.broadcasted_iota(jnp.int32, (16, 128), 0)
    uu = lax.broadcasted_iota(jnp.int32, (16, 128), 1) // 16
    e0 = (hh == uu).astype(jnp.float32)
    e1 = (hh == uu + 8).astype(jnp.float32)
    dl0 = jnp.maximum(jnp.dot(den, e0, preferred_element_type=jnp.float32), 1e-20)
    dl1 = jnp.maximum(jnp.dot(den, e1, preferred_element_type=jnp.float32), 1e-20)
    x = (x0_ref[...] + jnp.maximum(a0_ref[...] / dl0, 0.0)
         + jnp.maximum(a1_ref[...] / dl1, 0.0))
    oh = (lax.broadcasted_iota(jnp.int32, (64, N), 0) == gid_ref[...]).astype(jnp.float32)
    pooled = jnp.dot(oh, x, preferred_element_type=jnp.float32)
    h = _selu(jnp.dot(pooled, wr1_ref[...], preferred_element_type=jnp.float32) + br1_ref[...])
    h = _selu(jnp.dot(h, wr2_ref[...], preferred_element_type=jnp.float32) + br2_ref[...])
    out_ref[...] = jnp.dot(h, wr3_ref[...], preferred_element_type=jnp.float32) + br3_ref[...]


def _fin(x0, a0, a1, d0, d1, gid, wr1, br1, wr2, br2, wr3, br3):
    return pl.pallas_call(
        _fin_body,
        out_shape=jax.ShapeDtypeStruct((64, 1), jnp.float32),
    )(x0, a0, a1, d0, d1, gid, wr1, br1, wr2, br2, wr3, br3)


@functools.partial(
    pl.kernel, mesh=_mesh,
    out_type=[
        jax.ShapeDtypeStruct((E, 16), jnp.float32),
        jax.ShapeDtypeStruct((2 * NP8, 128), jnp.float32),
    ],
    scratch_types=[
        pltpu.VMEM((EPW,), jnp.int32),
        pltpu.VMEM((EPW,), jnp.int32),
        pltpu.VMEM((CH,), jnp.int32),
        pltpu.VMEM((CH,), jnp.int32),
        pltpu.VMEM((CH,), jnp.int32),
        pltpu.VMEM((CH,), jnp.int32),
        pltpu.VMEM((CH,), jnp.int32),
        pltpu.VMEM((CH,), jnp.int32),
        pltpu.VMEM((CH,), jnp.int32),
        pltpu.VMEM((CH,), jnp.int32),
        pltpu.VMEM((CH, 128), jnp.float32),
        pltpu.VMEM((CH, 128), jnp.float32),
        pltpu.VMEM((CH, 128), jnp.float32),
        pltpu.VMEM((CH, 128), jnp.float32),
        pltpu.VMEM((CH, 16), jnp.float32),
        pltpu.VMEM((CH, 128), jnp.float32),
        pltpu.VMEM_SHARED((NP8, 128), jnp.float32),
        pltpu.SemaphoreType.DMA,
        pltpu.SemaphoreType.DMA,
        pltpu.SemaphoreType.DMA,
        pltpu.SemaphoreType.DMA,
    ],
)
def _pass_a(src_hbm, dst_hbm, apad_hbm, slin_hbm, dden_hbm,
            srcall, dstall, si0, si1, di0, di1, p0, p1, m0, m1,
            as0, as1, ad0, ad1, sv, spad, dsh, smA0, smA1, smB0, smB1):
    cid = lax.axis_index("c")
    sid = lax.axis_index("s")
    wid = sid * NC + cid
    tbase = wid * EPW

    def zloop(i, _):
        for k in range(8):
            spad[i, pl.ds(k * 16, 16)] = jnp.zeros((16,), jnp.float32)
        return 0
    lax.fori_loop(0, CH, zloop, 0)
    pltpu.sync_copy(spad, dsh.at[pl.ds(sid * RP8, RP8)])
    pltpu.sync_copy(src_hbm.at[pl.ds(tbase, EPW)], srcall)
    pltpu.sync_copy(dst_hbm.at[pl.ds(tbase, EPW)], dstall)
    plsc.subcore_barrier()

    slots = ((si0, di0, p0, m0, as0, ad0, smA0, smB0),
             (si1, di1, p1, m1, as1, ad1, smA1, smB1))

    def issue(c, b):
        si, di, pp, mm, asv, adv, smA, smB = slots[b]
        s0 = c * CH
        for k in range(CH // 16):
            sl = pl.ds(k * 16, 16)
            s16 = srcall[pl.ds(s0 + k * 16, 16)]
            si[sl] = s16
            pp[sl] = lax.shift_right_logical(s16, 3)
            mm[sl] = lax.bitwise_and(s16, 7)
            di[sl] = dstall[pl.ds(s0 + k * 16, 16)]
        pltpu.async_copy(apad_hbm.at[si], asv, smA)
        pltpu.async_copy(apad_hbm.at[di], adv, smB)

    def finish(c, b):
        si, di, pp, mm, asv, adv, smA, smB = slots[b]
        pltpu.make_async_copy(apad_hbm.at[si], asv, smA).wait()
        pltpu.make_async_copy(apad_hbm.at[di], adv, smB).wait()
        z16 = jnp.zeros((16,), jnp.float32)

        def body(j, _):
            mrow = mm[pl.ds(j * 16, 16)]
            for l in range(16):
                i = j * 16 + l
                v = asv[i, pl.ds(0, 16)] + adv[i, pl.ds(16, 16)]
                v = jnp.where(v >= 0.0, v, 0.2 * v)
                v = jnp.clip(v, -2.0, 2.0)
                s = jnp.exp(v)
                sv[i, :] = s
                mk = mrow[l]
                for k in range(8):
                    spad[i, pl.ds(k * 16, 16)] = jnp.where(mk == k, s, z16)
            return 0
        lax.fori_loop(0, CH // 16, body, 0)
        pltpu.sync_copy(sv, slin_hbm.at[pl.ds(tbase + c * CH, CH)])
        pltpu.sync_copy(spad, dsh.at[pp], add=True)

    issue(0, 0)

    def outer(g, _):
        c0 = 2 * g

        @pl.when(c0 + 1 < NCHA)
        def _():
            issue(c0 + 1, 1)
        finish(c0, 0)

        @pl.when(c0 + 2 < NCHA)
        def _():
            issue(c0 + 2, 0)

        @pl.when(c0 + 1 < NCHA)
        def _():
            finish(c0 + 1, 1)
        return 0
    lax.fori_loop(0, (NCHA + 1) // 2, outer, 0)

    plsc.subcore_barrier()
    pltpu.sync_copy(dsh.at[pl.ds(sid * RP8, RP8)],
                    dden_hbm.at[pl.ds(cid * NP8 + sid * RP8, RP8)])


@functools.partial(
    pl.kernel, mesh=_mesh,
    out_type=jax.ShapeDtypeStruct((2 * NP, 128), jnp.float32),
    scratch_types=[
        pltpu.VMEM((SEG,), jnp.int32),
        pltpu.VMEM((SEG,), jnp.int32),
        pltpu.VMEM((CH,), jnp.int32),
        pltpu.VMEM((CH,), jnp.int32),
        pltpu.VMEM((CH,), jnp.int32),
        pltpu.VMEM((CH,), jnp.int32),
        pltpu.VMEM((CH, 16), jnp.float32),
        pltpu.VMEM((CH, 16), jnp.float32),
        pltpu.VMEM((CH, 128), jnp.float32),
        pltpu.VMEM((CH, 128), jnp.float32),
        pltpu.VMEM_SHARED((NP, 128), jnp.float32),
        pltpu.SemaphoreType.DMA,
        pltpu.SemaphoreType.DMA,
        pltpu.SemaphoreType.DMA,
        pltpu.SemaphoreType.DMA,
    ],
)
def _pass_c(src_hbm, dst2_hbm, slin_hbm, nst2_hbm, out_hbm,
            srcseg, dstseg, si0, si1, gi0, gi1, sv0, sv1, rw0, rw1,
            accsh, smS0, smS1, smG0, smG1):
    cid = lax.axis_index("c")
    sid = lax.axis_index("s")
    tbase = sid * EPT

    def zloop(i, _):
        for k in range(8):
            rw0[i, pl.ds(k * 16, 16)] = jnp.zeros((16,), jnp.float32)
        return 0
    lax.fori_loop(0, CH, zloop, 0)
    for j in range(7):
        pltpu.sync_copy(rw0, accsh.at[pl.ds(sid * RPT + j * 80, 80)])
    pltpu.sync_copy(rw0.at[pl.ds(0, 72)], accsh.at[pl.ds(sid * RPT + 560, 72)])
    plsc.subcore_barrier()

    is_c0 = cid == 0
    slots = ((si0, gi0, sv0, rw0, smS0, smG0), (si1, gi1, sv1, rw1, smS1, smG1))

    def finish(ebase, c, b):
        si, gi, sv, rows, smS, smG = slots[b]
        pltpu.make_async_copy(slin_hbm.at[pl.ds(ebase + c * CH, CH)], sv, smS).wait()
        pltpu.make_async_copy(nst2_hbm.at[gi], rows, smG).wait()

        def body(i, _):
            srow = sv[i, :]
            for h in range(8):
                wsc = jnp.where(is_c0, srow[h], srow[8 + h])
                rows[i, pl.ds(h * 16, 16)] = rows[i, pl.ds(h * 16, 16)] * wsc
            return 0
        lax.fori_loop(0, CH, body, 0)
        pltpu.sync_copy(rows, accsh.at[si], add=True)

    for g in range(NSEG):
        ebase = tbase + g * SEG
        pltpu.sync_copy(src_hbm.at[pl.ds(ebase, SEG)], srcseg)
        pltpu.sync_copy(dst2_hbm.at[pl.ds(cid * E + ebase, SEG)], dstseg)

        def issue(c, b):
            si, gi, sv, rows, smS, smG = slots[b]
            s0 = c * CH
            for k in range(CH // 16):
                sl = pl.ds(k * 16, 16)
                si[sl] = srcseg[pl.ds(s0 + k * 16, 16)]
                gi[sl] = dstseg[pl.ds(s0 + k * 16, 16)]
            pltpu.async_copy(slin_hbm.at[pl.ds(ebase + s0, CH)], sv, smS)
            pltpu.async_copy(nst2_hbm.at[gi], rows, smG)

        issue(0, 0)

        def outer(t, _):
            c0 = 2 * t
            issue(c0 + 1, 1)
            finish(ebase, c0, 0)

            @pl.when(c0 + 2 < NCHS)
            def _():
                issue(c0 + 2, 0)
            finish(ebase, c0 + 1, 1)
            return 0
        lax.fori_loop(0, NCHS // 2, outer, 0)

    plsc.subcore_barrier()
    pltpu.sync_copy(accsh.at[pl.ds(sid * RPT, RPT)],
                    out_hbm.at[pl.ds(cid * NP + sid * RPT, RPT)])


def kernel(states_action, states_graph_ids, states_first, states_second,
           ordered_edges, W0, b0, att_kernels, att_attn,
           Wr1, br1, Wr2, br2, Wr3, br3):
    f32 = jnp.float32
    kcat = jnp.transpose(att_kernels, (2, 0, 1, 3)).reshape(128, 256)
    kas = att_attn[:, :, :16, 0].reshape(1, 256).astype(f32)
    kad = att_attn[:, :, 16:, 0].reshape(1, 256).astype(f32)

    x0, nst_all, apad = _pre(states_action, W0, b0.reshape(1, 128), kcat, kas, kad)

    src = states_first.astype(jnp.int32)
    dst = states_second.astype(jnp.int32)

    slin, dden = _pass_a(src, dst, apad)
    pad = jnp.zeros((NP - N, 128), f32)
    nst2 = jnp.concatenate([nst_all[:, :128], pad, nst_all[:, 128:], pad], axis=0)
    dst2 = jnp.concatenate([dst, dst + NP])
    acc = _pass_c(src, dst2, slin, nst2)

    gid = states_graph_ids.astype(jnp.int32).reshape(1, N)
    d0 = dden[:NP8].reshape(NP8 * 8, 16)[:N]
    d1 = dden[NP8:].reshape(NP8 * 8, 16)[:N]
    return _fin(x0, acc[:N], acc[NP:NP + N], d0, d1, gid,
                Wr1, br1.reshape(1, 35), Wr2, br2.reshape(1, 35),
                Wr3, br3.reshape(1, 1))
